# baseline (speedup 1.0000x reference)
import functools

import jax
import jax.numpy as jnp
from jax import lax
from jax.experimental import pallas as pl
from jax.experimental.pallas import tpu as pltpu
from jax.experimental.pallas import tpu_sc as plsc

EMB = 2048
NE = 16
NTOK = 16384
TC_BLK = 2048
NW = 32
TOK_PER_W = NTOK // NW
LANES = 16
GROUPS = TOK_PER_W // LANES


def _tc_scores_body(x_ref, w_ref, b_ref, out_ref):
    s = lax.dot_general(
        w_ref[...], x_ref[...], (((1,), (1,)), ((), ())),
        preferred_element_type=jnp.float32,
        precision=lax.Precision.DEFAULT,
    )
    out_ref[...] = s + b_ref[...]


def _tc_scores(x, W, b2):
    return pl.pallas_call(
        _tc_scores_body,
        grid=(NTOK // TC_BLK,),
        in_specs=[
            pl.BlockSpec((TC_BLK, EMB), lambda i: (i, 0)),
            pl.BlockSpec((NE, EMB), lambda i: (0, 0)),
            pl.BlockSpec((NE, 1), lambda i: (0, 0)),
        ],
        out_specs=pl.BlockSpec((NE, TC_BLK), lambda i: (0, i)),
        out_shape=jax.ShapeDtypeStruct((NE, NTOK), jnp.float32),
        compiler_params=pltpu.CompilerParams(
            dimension_semantics=("arbitrary",)),
    )(x, W, b2)


def _sc_topk(scores):
    mesh = plsc.VectorSubcoreMesh(core_axis_name="c", subcore_axis_name="s")

    @functools.partial(
        pl.kernel,
        mesh=mesh,
        out_type=[
            jax.ShapeDtypeStruct((NTOK,), jnp.float32),
            jax.ShapeDtypeStruct((NTOK,), jnp.float32),
            jax.ShapeDtypeStruct((NTOK,), jnp.int32),
            jax.ShapeDtypeStruct((NTOK,), jnp.int32),
        ],
        scratch_types=[
            pltpu.VMEM((NE, TOK_PER_W), jnp.float32),
            pltpu.VMEM((TOK_PER_W,), jnp.float32),
            pltpu.VMEM((TOK_PER_W,), jnp.float32),
            pltpu.VMEM((TOK_PER_W,), jnp.int32),
            pltpu.VMEM((TOK_PER_W,), jnp.int32),
        ],
    )
    def k(scores_hbm, v1_hbm, v2_hbm, i1_hbm, i2_hbm,
          sc_v, v1_v, v2_v, i1_v, i2_v):
        wid = lax.axis_index("s") * 2 + lax.axis_index("c")
        base = wid * TOK_PER_W
        for e in range(NE):
            pltpu.sync_copy(scores_hbm.at[e, pl.ds(base, TOK_PER_W)],
                            sc_v.at[e])

        def group(g, carry):
            col = pl.ds(pl.multiple_of(g * LANES, LANES), LANES)
            v0 = sc_v[0, col]
            m1 = v0
            i1 = jnp.zeros((LANES,), jnp.int32)
            for e in range(1, NE):
                ve = sc_v[e, col]
                gt = ve > m1
                m1 = jnp.where(gt, ve, m1)
                i1 = jnp.where(gt, e, i1)
            neg = jnp.float32(-jnp.inf)
            m2 = jnp.where(i1 == 0, neg, v0)
            i2 = jnp.zeros((LANES,), jnp.int32)
            for e in range(1, NE):
                ve = jnp.where(i1 == e, neg, sc_v[e, col])
                gt = ve > m2
                m2 = jnp.where(gt, ve, m2)
                i2 = jnp.where(gt, e, i2)
            e2 = jnp.exp(m2 - m1)
            den = e2 + jnp.float32(1.0)
            v1_v[col] = jnp.float32(1.0) / den
            v2_v[col] = e2 / den
            i1_v[col] = i1
            i2_v[col] = i2
            return carry

        lax.fori_loop(0, GROUPS, group, 0)
        sl = pl.ds(base, TOK_PER_W)
        pltpu.sync_copy(v1_v, v1_hbm.at[sl])
        pltpu.sync_copy(v2_v, v2_hbm.at[sl])
        pltpu.sync_copy(i1_v, i1_hbm.at[sl])
        pltpu.sync_copy(i2_v, i2_hbm.at[sl])

    return k(scores)


def kernel(x, W, b):
    scores = _tc_scores(x, W, jnp.reshape(b, (NE, 1)))
    v1, v2, i1, i2 = _sc_topk(scores)
    values = jnp.stack([v1, v2], axis=-1)
    indices = jnp.stack([i1, i2], axis=-1)
    return values, indices


# speedup vs baseline: 1.1119x; 1.1119x over previous
import functools

import jax
import jax.numpy as jnp
from jax import lax
from jax.experimental import pallas as pl
from jax.experimental.pallas import tpu as pltpu
from jax.experimental.pallas import tpu_sc as plsc

EMB = 2048
NE = 16
NTOK = 16384
TC_BLK = 2048
SUB = TC_BLK // 512
NW = 32
TOK_PER_W = NTOK // NW
LANES = 16
GROUPS = TOK_PER_W // LANES


def _tc_scores_body(x_ref, w_ref, b_ref, out_ref):
    for j in range(SUB):
        s = lax.dot_general(
            w_ref[...], x_ref[pl.ds(j * 512, 512), :],
            (((1,), (1,)), ((), ())),
            preferred_element_type=jnp.float32,
            precision=lax.Precision.DEFAULT,
        )
        out_ref[j] = s + b_ref[...]


def _tc_scores(x, W, b2):
    return pl.pallas_call(
        _tc_scores_body,
        grid=(NTOK // TC_BLK,),
        in_specs=[
            pl.BlockSpec((TC_BLK, EMB), lambda i: (i, 0)),
            pl.BlockSpec((NE, EMB), lambda i: (0, 0)),
            pl.BlockSpec((NE, 1), lambda i: (0, 0)),
        ],
        out_specs=pl.BlockSpec((SUB, NE, 512), lambda i: (i, 0, 0)),
        out_shape=jax.ShapeDtypeStruct((NW, NE, TOK_PER_W), jnp.float32),
        compiler_params=pltpu.CompilerParams(
            dimension_semantics=("arbitrary",)),
    )(x, W, b2)


def _sc_topk(scores):
    mesh = plsc.VectorSubcoreMesh(core_axis_name="c", subcore_axis_name="s")

    @functools.partial(
        pl.kernel,
        mesh=mesh,
        out_type=[
            jax.ShapeDtypeStruct((NTOK,), jnp.float32),
            jax.ShapeDtypeStruct((NTOK,), jnp.float32),
            jax.ShapeDtypeStruct((NTOK,), jnp.int32),
            jax.ShapeDtypeStruct((NTOK,), jnp.int32),
        ],
        scratch_types=[
            pltpu.VMEM((NE, TOK_PER_W), jnp.float32),
            pltpu.VMEM((TOK_PER_W,), jnp.float32),
            pltpu.VMEM((TOK_PER_W,), jnp.float32),
            pltpu.VMEM((TOK_PER_W,), jnp.int32),
            pltpu.VMEM((TOK_PER_W,), jnp.int32),
        ],
    )
    def k(scores_hbm, v1_hbm, v2_hbm, i1_hbm, i2_hbm,
          sc_v, v1_v, v2_v, i1_v, i2_v):
        wid = lax.axis_index("s") * 2 + lax.axis_index("c")
        pltpu.sync_copy(scores_hbm.at[wid], sc_v)

        def group(g, carry):
            col = pl.ds(pl.multiple_of(g * LANES, LANES), LANES)
            v0 = sc_v[0, col]
            m1 = v0
            i1 = jnp.zeros((LANES,), jnp.int32)
            for e in range(1, NE):
                ve = sc_v[e, col]
                gt = ve > m1
                m1 = jnp.where(gt, ve, m1)
                i1 = jnp.where(gt, e, i1)
            neg = jnp.float32(-jnp.inf)
            m2 = jnp.where(i1 == 0, neg, v0)
            i2 = jnp.zeros((LANES,), jnp.int32)
            for e in range(1, NE):
                ve = jnp.where(i1 == e, neg, sc_v[e, col])
                gt = ve > m2
                m2 = jnp.where(gt, ve, m2)
                i2 = jnp.where(gt, e, i2)
            e2 = jnp.exp(m2 - m1)
            den = e2 + jnp.float32(1.0)
            v1_v[col] = jnp.float32(1.0) / den
            v2_v[col] = e2 / den
            i1_v[col] = i1
            i2_v[col] = i2
            return carry

        lax.fori_loop(0, GROUPS, group, 0)
        sl = pl.ds(wid * TOK_PER_W, TOK_PER_W)
        pltpu.sync_copy(v1_v, v1_hbm.at[sl])
        pltpu.sync_copy(v2_v, v2_hbm.at[sl])
        pltpu.sync_copy(i1_v, i1_hbm.at[sl])
        pltpu.sync_copy(i2_v, i2_hbm.at[sl])

    return k(scores)


def kernel(x, W, b):
    scores = _tc_scores(x, W, jnp.reshape(b, (NE, 1)))
    v1, v2, i1, i2 = _sc_topk(scores)
    values = jnp.stack([v1, v2], axis=-1)
    indices = jnp.stack([i1, i2], axis=-1)
    return values, indices
